# baseline (speedup 1.0000x reference)
import dataclasses
import functools

import jax
import jax.numpy as jnp
from jax import lax
from jax.experimental import pallas as pl
from jax.experimental.pallas import tpu as pltpu
from jax.experimental.pallas import tpu_sc as plsc

VOCAB_PAD = 1024
NC, NS, L = 2, 16, 16
NW = NC * NS


def _project_table_kernel(tt_ref, wt_ref, b_ref, out_ref, *, inv_seq, vocab):
    dot = lax.dot_general(
        wt_ref[...], tt_ref[...],
        dimension_numbers=(((1,), (0,)), ((), ())),
        preferred_element_type=jnp.float32,
        precision=lax.Precision.HIGHEST,
    )
    out_ref[...] = jnp.zeros_like(out_ref)
    out_ref[:, :vocab] = (dot + b_ref[0]) * inv_seq


def _make_sc_gather_pool(batch, seq):
    rows_per_w = batch // NW
    chunks = rows_per_w // L
    mesh = plsc.VectorSubcoreMesh(core_axis_name="c", subcore_axis_name="s")

    cp = pltpu.CompilerParams()
    if "needs_layout_passes" in pltpu.CompilerParams.__dataclass_fields__:
        cp = dataclasses.replace(cp, needs_layout_passes=False)

    @functools.partial(
        pl.kernel,
        compiler_params=cp,
        out_type=jax.ShapeDtypeStruct((batch,), jnp.float32),
        mesh=mesh,
        scratch_types=[
            pltpu.VMEM((seq, rows_per_w), jnp.int32),
            pltpu.VMEM((VOCAB_PAD,), jnp.float32),
            pltpu.VMEM((rows_per_w,), jnp.float32),
            pltpu.SemaphoreType.DMA,
            pltpu.SemaphoreType.DMA,
        ],
    )
    def sc_kernel(xt_hbm, t_hbm, o_hbm, x_v, t_v, o_v, sem_t, sem_x):
        wid = lax.axis_index("s") * NC + lax.axis_index("c")
        base = wid * rows_per_w
        cp_t = pltpu.async_copy(t_hbm.at[0], t_v, sem_t)
        cp_x = pltpu.async_copy(xt_hbm.at[:, pl.ds(base, rows_per_w)], x_v,
                                sem_x)
        cp_t.wait()
        cp_x.wait()

        @plsc.parallel_loop(0, chunks, unroll=2)
        def _(c):
            row0 = c * L

            def body(l, acc):
                xv = x_v[l, pl.ds(row0, L)]
                return acc + plsc.load_gather(t_v, [xv])

            acc = lax.fori_loop(0, seq, body, jnp.zeros((L,), jnp.float32),
                                unroll=5)
            o_v[pl.ds(row0, L)] = 1.0 / (1.0 + jnp.exp(-acc))

        pltpu.sync_copy(o_v, o_hbm.at[pl.ds(base, rows_per_w)])

    return sc_kernel


def kernel(x, table, W, b):
    batch, seq = x.shape
    vocab, dim = table.shape

    t_proj = pl.pallas_call(
        functools.partial(_project_table_kernel, inv_seq=1.0 / seq,
                          vocab=vocab),
        out_shape=jax.ShapeDtypeStruct((1, VOCAB_PAD), jnp.float32),
        in_specs=[
            pl.BlockSpec(memory_space=pltpu.VMEM),
            pl.BlockSpec(memory_space=pltpu.VMEM),
            pl.BlockSpec(memory_space=pltpu.SMEM),
        ],
        out_specs=pl.BlockSpec(memory_space=pltpu.VMEM),
    )(table.T, W.T, b)

    out_flat = _make_sc_gather_pool(batch, seq)(x.T.astype(jnp.int32), t_proj)
    return out_flat.reshape(batch, 1)


# speedup vs baseline: 1.0677x; 1.0677x over previous
import dataclasses
import functools

import jax
import jax.numpy as jnp
from jax import lax
from jax.experimental import pallas as pl
from jax.experimental.pallas import tpu as pltpu
from jax.experimental.pallas import tpu_sc as plsc

VOCAB_PAD = 1024
NC, NS, L = 2, 16, 16
NW = NC * NS


def _project_table_kernel(tt_ref, wt_ref, b_ref, out_ref, *, inv_seq, vocab):
    dot = lax.dot_general(
        wt_ref[...], tt_ref[...],
        dimension_numbers=(((1,), (0,)), ((), ())),
        preferred_element_type=jnp.float32,
        precision=lax.Precision.HIGHEST,
    )
    out_ref[...] = jnp.zeros_like(out_ref)
    out_ref[:, :vocab] = (dot + b_ref[0]) * inv_seq


def _make_sc_gather_pool(batch, seq):
    rows_per_w = batch // NW
    chunks = rows_per_w // L
    mesh = plsc.VectorSubcoreMesh(core_axis_name="c", subcore_axis_name="s")

    cp = pltpu.CompilerParams()
    if "needs_layout_passes" in pltpu.CompilerParams.__dataclass_fields__:
        cp = dataclasses.replace(cp, needs_layout_passes=False)

    @functools.partial(
        pl.kernel,
        compiler_params=cp,
        out_type=jax.ShapeDtypeStruct((batch,), jnp.float32),
        mesh=mesh,
        scratch_types=[
            pltpu.VMEM((seq, rows_per_w), jnp.int32),
            pltpu.VMEM((VOCAB_PAD,), jnp.float32),
            pltpu.VMEM((rows_per_w,), jnp.float32),
            pltpu.SemaphoreType.DMA,
            pltpu.SemaphoreType.DMA,
        ],
    )
    def sc_kernel(xt_hbm, t_hbm, o_hbm, x_v, t_v, o_v, sem_t, sem_x):
        wid = lax.axis_index("s") * NC + lax.axis_index("c")
        base = wid * rows_per_w
        cp_t = pltpu.async_copy(t_hbm.at[0], t_v, sem_t)
        cp_x = pltpu.async_copy(xt_hbm.at[:, pl.ds(base, rows_per_w)], x_v,
                                sem_x)
        cp_t.wait()
        cp_x.wait()

        @pl.loop(0, chunks)
        def _(c):
            row0 = c * L

            def body(l, acc):
                xv = x_v[l, pl.ds(row0, L)]
                return acc + plsc.load_gather(t_v, [xv])

            acc = lax.fori_loop(0, seq, body, jnp.zeros((L,), jnp.float32),
                                unroll=5)
            o_v[pl.ds(row0, L)] = 1.0 / (1.0 + jnp.exp(-acc))

        pltpu.sync_copy(o_v, o_hbm.at[pl.ds(base, rows_per_w)])

    return sc_kernel


def kernel(x, table, W, b):
    batch, seq = x.shape
    vocab, dim = table.shape

    t_proj = pl.pallas_call(
        functools.partial(_project_table_kernel, inv_seq=1.0 / seq,
                          vocab=vocab),
        out_shape=jax.ShapeDtypeStruct((1, VOCAB_PAD), jnp.float32),
        in_specs=[
            pl.BlockSpec(memory_space=pltpu.VMEM),
            pl.BlockSpec(memory_space=pltpu.VMEM),
            pl.BlockSpec(memory_space=pltpu.SMEM),
        ],
        out_specs=pl.BlockSpec(memory_space=pltpu.VMEM),
    )(table.T, W.T, b)

    out_flat = _make_sc_gather_pool(batch, seq)(x.T.astype(jnp.int32), t_proj)
    return out_flat.reshape(batch, 1)
